# baseline (speedup 1.0000x reference)
import functools

import jax
import jax.numpy as jnp
from jax import lax
from jax.experimental import pallas as pl
from jax.experimental.pallas import tpu as pltpu
from jax.experimental.pallas import tpu_sc as plsc

B = 4
L = 8192
C = 768
K = 8
LANES = 16
NCG = C // LANES
NTASK = B * NCG
NW = 32
TPW = NTASK // NW
LC = 1024
NCHUNK = L // LC


G = 8

_SORT8 = (
    (0, 1), (2, 3), (4, 5), (6, 7),
    (0, 2), (1, 3), (4, 6), (5, 7),
    (1, 2), (5, 6),
    (0, 4), (1, 5), (2, 6), (3, 7),
    (2, 4), (3, 5),
    (1, 2), (3, 4), (5, 6),
)
_BMERGE = (
    (0, 4), (1, 5), (2, 6), (3, 7),
    (0, 2), (1, 3), (4, 6), (5, 7),
    (0, 1), (2, 3), (4, 5), (6, 7),
)


def _inner(buf, V):
    @plsc.parallel_loop(0, LC // G, unroll=4, carry=V)
    def group_body(i, V):
        base = i * G
        rows = [buf[base + j] for j in range(G)]
        for a, b in _SORT8:
            hi = jnp.maximum(rows[a], rows[b])
            lo = jnp.minimum(rows[a], rows[b])
            rows[a], rows[b] = hi, lo
        M = [jnp.maximum(V[k], rows[K - 1 - k]) for k in range(K)]
        for a, b in _BMERGE:
            hi = jnp.maximum(M[a], M[b])
            lo = jnp.minimum(M[a], M[b])
            M[a], M[b] = hi, lo
        return tuple(M)

    return group_body


@functools.partial(
    pl.kernel,
    mesh=plsc.VectorSubcoreMesh(core_axis_name="c", subcore_axis_name="s"),
    out_type=jax.ShapeDtypeStruct((B, K, C), jnp.float32),
    scratch_types=[
        pltpu.VMEM((LC, LANES), jnp.float32),
        pltpu.VMEM((LC, LANES), jnp.float32),
        pltpu.VMEM((K, LANES), jnp.float32),
        pltpu.SemaphoreType.DMA,
        pltpu.SemaphoreType.DMA,
    ],
    compiler_params=pltpu.CompilerParams(
        use_tc_tiling_on_sc=False, needs_layout_passes=False
    ),
)
def _topk_sc(x_hbm, out_hbm, buf0, buf1, outb, sem0, sem1):
    wid = lax.axis_index("s") * 2 + lax.axis_index("c")
    bufs = (buf0, buf1)
    sems = (sem0, sem1)

    for t in range(TPW):
        g = t * NW + wid
        b = g // NCG
        cg = g - b * NCG
        c0 = cg * LANES

        def src(chunk, b=b, c0=c0):
            return x_hbm.at[b, pl.ds(chunk * LC, LC), pl.ds(c0, LANES)]

        def start(chunk, slot):
            pltpu.async_copy(src(chunk), bufs[slot], sems[slot])

        def wait(chunk, slot):
            pltpu.make_async_copy(src(chunk), bufs[slot], sems[slot]).wait()

        start(0, 0)
        neg_inf = jnp.full((LANES,), -jnp.inf, dtype=jnp.float32)
        V = tuple(neg_inf for _ in range(K))

        def pair_body(p, V):
            c = 2 * p
            start(c + 1, 1)
            wait(c, 0)
            V = _inner(buf0, V)

            @pl.when(p < NCHUNK // 2 - 1)
            def _():
                start(c + 2, 0)

            wait(c + 1, 1)
            V = _inner(buf1, V)
            return V

        V = lax.fori_loop(0, NCHUNK // 2, pair_body, V)

        for kk in range(K):
            outb[kk] = V[kk]
        pltpu.sync_copy(outb, out_hbm.at[b, pl.ds(0, K), pl.ds(c0, LANES)])


def kernel(inputs):
    return _topk_sc(inputs)


# speedup vs baseline: 1.7303x; 1.7303x over previous
import functools

import jax
import jax.numpy as jnp
from jax import lax
from jax.experimental import pallas as pl
from jax.experimental.pallas import tpu as pltpu
from jax.experimental.pallas import tpu_sc as plsc

B = 4
L = 8192
C = 768
K = 8
LANES = 16
NCB = 6
SUBS = 8
TPB = 8
LPT = L // TPB
LC = 32
NCHUNK = LPT // LC

_SORT8 = (
    (0, 1), (2, 3), (4, 5), (6, 7),
    (0, 2), (1, 3), (4, 6), (5, 7),
    (1, 2), (5, 6),
    (0, 4), (1, 5), (2, 6), (3, 7),
    (2, 4), (3, 5),
    (1, 2), (3, 4), (5, 6),
)
_BMERGE = (
    (0, 4), (1, 5), (2, 6), (3, 7),
    (0, 2), (1, 3), (4, 6), (5, 7),
    (0, 1), (2, 3), (4, 5), (6, 7),
)


def _merge_sorted(A, Bs):
    M = [jnp.maximum(A[k], Bs[K - 1 - k]) for k in range(K)]
    for a, b in _BMERGE:
        hi = jnp.maximum(M[a], M[b])
        lo = jnp.minimum(M[a], M[b])
        M[a], M[b] = hi, lo
    return M


@functools.partial(
    pl.kernel,
    mesh=plsc.VectorSubcoreMesh(core_axis_name="c", subcore_axis_name="s"),
    out_type=jax.ShapeDtypeStruct((B, K, C), jnp.float32),
    scratch_types=[
        pltpu.VMEM((LC, C), jnp.float32),
        pltpu.VMEM((LC, C), jnp.float32),
        pltpu.VMEM((NCB, K, 128), jnp.float32),
        pltpu.VMEM((TPB, K, 128), jnp.float32),
        pltpu.VMEM((K, 128), jnp.float32),
        pltpu.VMEM_SHARED((NCB, 2 * TPB, K, 128), jnp.float32),
        pltpu.SemaphoreType.DMA,
        pltpu.SemaphoreType.DMA,
    ],
    compiler_params=pltpu.CompilerParams(
        use_tc_tiling_on_sc=True, needs_layout_passes=False
    ),
)
def _topk_sc(x_hbm, out_hbm, buf0, buf1, state, mbuf, outv, S, sem0, sem1):
    core = lax.axis_index("c")
    sid = lax.axis_index("s")
    grp = sid // TPB
    b = 2 * core + grp
    w = sid - grp * TPB
    l0 = w * LPT

    bufs = (buf0, buf1)
    sems = (sem0, sem1)

    def src(chunk):
        return x_hbm.at[b, pl.ds(l0 + chunk * LC, LC)]

    def start(chunk, slot):
        pltpu.async_copy(src(chunk), bufs[slot], sems[slot])

    def wait(chunk, slot):
        pltpu.make_async_copy(src(chunk), bufs[slot], sems[slot]).wait()

    neg_inf = jnp.full((LANES,), -jnp.inf, dtype=jnp.float32)

    @plsc.parallel_loop(0, NCB * K)
    def _init(i):
        cb = i // K
        k = i - cb * K
        for j in range(SUBS):
            state[cb, k, pl.ds(j * LANES, LANES)] = neg_inf

    def process(buf):
        for cb in range(NCB):
            def jbody(j, carry, buf=buf, cb=cb):
                c0 = cb * 128 + j * LANES
                js = j * LANES
                V = tuple(state[cb, k, pl.ds(js, LANES)] for k in range(K))

                @plsc.parallel_loop(0, LC // 8, carry=V)
                def mloop(m, V):
                    rows = [buf[m * 8 + s, pl.ds(c0, LANES)] for s in range(8)]
                    for a, bb in _SORT8:
                        hi = jnp.maximum(rows[a], rows[bb])
                        lo = jnp.minimum(rows[a], rows[bb])
                        rows[a], rows[bb] = hi, lo
                    return tuple(_merge_sorted(V, rows))

                for k in range(K):
                    state[cb, k, pl.ds(js, LANES)] = mloop[k]
                return carry

            lax.fori_loop(0, SUBS, jbody, 0)

    start(0, 0)

    def pair_body(p, carry):
        ch = 2 * p
        start(ch + 1, 1)
        wait(ch, 0)
        process(buf0)

        @pl.when(p < NCHUNK // 2 - 1)
        def _():
            start(ch + 2, 0)

        wait(ch + 1, 1)
        process(buf1)
        return carry

    lax.fori_loop(0, NCHUNK // 2, pair_body, 0)

    for cb in range(NCB):
        pltpu.sync_copy(state.at[cb], S.at[cb, sid])
    plsc.subcore_barrier()

    mcb = sid - grp * TPB

    @pl.when(mcb < NCB)
    def _():
        pltpu.sync_copy(S.at[mcb, pl.ds(grp * TPB, TPB)], mbuf)

        def sub_body(sub, carry):
            ss = sub * LANES
            cur = [
                [mbuf[wv, k, pl.ds(ss, LANES)] for k in range(K)]
                for wv in range(TPB)
            ]
            while len(cur) > 1:
                cur = [
                    _merge_sorted(cur[2 * i], cur[2 * i + 1])
                    for i in range(len(cur) // 2)
                ]
            fin = cur[0]
            for k in range(K):
                outv[k, pl.ds(ss, LANES)] = fin[k]
            return carry

        lax.fori_loop(0, SUBS, sub_body, 0)
        pltpu.sync_copy(
            outv, out_hbm.at[b, pl.ds(0, K), pl.ds(mcb * 128, 128)]
        )


def kernel(inputs):
    return _topk_sc(inputs)


# speedup vs baseline: 1.8146x; 1.0487x over previous
import functools

import jax
import jax.numpy as jnp
from jax import lax
from jax.experimental import pallas as pl
from jax.experimental.pallas import tpu as pltpu
from jax.experimental.pallas import tpu_sc as plsc

B = 4
L = 8192
C = 768
K = 8
LANES = 16
NCB = 6
SUBS = 8
TPB = 8
LPT = L // TPB
LC = 64
NCHUNK = LPT // LC

_SORT8 = (
    (0, 1), (2, 3), (4, 5), (6, 7),
    (0, 2), (1, 3), (4, 6), (5, 7),
    (1, 2), (5, 6),
    (0, 4), (1, 5), (2, 6), (3, 7),
    (2, 4), (3, 5),
    (1, 2), (3, 4), (5, 6),
)
_BMERGE = (
    (0, 4), (1, 5), (2, 6), (3, 7),
    (0, 2), (1, 3), (4, 6), (5, 7),
    (0, 1), (2, 3), (4, 5), (6, 7),
)


def _merge_sorted(A, Bs):
    M = [jnp.maximum(A[k], Bs[K - 1 - k]) for k in range(K)]
    for a, b in _BMERGE:
        hi = jnp.maximum(M[a], M[b])
        lo = jnp.minimum(M[a], M[b])
        M[a], M[b] = hi, lo
    return M


@functools.partial(
    pl.kernel,
    mesh=plsc.VectorSubcoreMesh(core_axis_name="c", subcore_axis_name="s"),
    out_type=jax.ShapeDtypeStruct((B, K, C), jnp.float32),
    scratch_types=[
        pltpu.VMEM((LC, C), jnp.float32),
        pltpu.VMEM((LC, C), jnp.float32),
        pltpu.VMEM((NCB, K, 128), jnp.float32),
        pltpu.VMEM((TPB, K, 128), jnp.float32),
        pltpu.VMEM((K, 128), jnp.float32),
        pltpu.VMEM_SHARED((NCB, 2 * TPB, K, 128), jnp.float32),
        pltpu.SemaphoreType.DMA,
        pltpu.SemaphoreType.DMA,
    ],
    compiler_params=pltpu.CompilerParams(
        use_tc_tiling_on_sc=True, needs_layout_passes=False
    ),
)
def _topk_sc(x_hbm, out_hbm, buf0, buf1, state, mbuf, outv, S, sem0, sem1):
    core = lax.axis_index("c")
    sid = lax.axis_index("s")
    grp = sid // TPB
    b = 2 * core + grp
    w = sid - grp * TPB
    l0 = w * LPT

    bufs = (buf0, buf1)
    sems = (sem0, sem1)

    def src(chunk):
        return x_hbm.at[b, pl.ds(l0 + chunk * LC, LC)]

    def start(chunk, slot):
        pltpu.async_copy(src(chunk), bufs[slot], sems[slot])

    def wait(chunk, slot):
        pltpu.make_async_copy(src(chunk), bufs[slot], sems[slot]).wait()

    neg_inf = jnp.full((LANES,), -jnp.inf, dtype=jnp.float32)

    @plsc.parallel_loop(0, NCB * K)
    def _init(i):
        cb = i // K
        k = i - cb * K
        for j in range(SUBS):
            state[cb, k, pl.ds(j * LANES, LANES)] = neg_inf

    def process(buf):
        for cb in range(NCB):
            def jbody(j, carry, buf=buf, cb=cb):
                c0 = cb * 128 + j * LANES
                js = j * LANES
                V = tuple(state[cb, k, pl.ds(js, LANES)] for k in range(K))

                @plsc.parallel_loop(0, LC // 8, carry=V)
                def mloop(m, V):
                    rows = [buf[m * 8 + s, pl.ds(c0, LANES)] for s in range(8)]
                    for a, bb in _SORT8:
                        hi = jnp.maximum(rows[a], rows[bb])
                        lo = jnp.minimum(rows[a], rows[bb])
                        rows[a], rows[bb] = hi, lo
                    return tuple(_merge_sorted(V, rows))

                for k in range(K):
                    state[cb, k, pl.ds(js, LANES)] = mloop[k]
                return carry

            lax.fori_loop(0, SUBS, jbody, 0)

    start(0, 0)

    def pair_body(p, carry):
        ch = 2 * p
        start(ch + 1, 1)
        wait(ch, 0)
        process(buf0)

        @pl.when(p < NCHUNK // 2 - 1)
        def _():
            start(ch + 2, 0)

        wait(ch + 1, 1)
        process(buf1)
        return carry

    lax.fori_loop(0, NCHUNK // 2, pair_body, 0)

    for cb in range(NCB):
        pltpu.sync_copy(state.at[cb], S.at[cb, sid])
    plsc.subcore_barrier()

    mcb = sid - grp * TPB

    @pl.when(mcb < NCB)
    def _():
        pltpu.sync_copy(S.at[mcb, pl.ds(grp * TPB, TPB)], mbuf)

        def sub_body(sub, carry):
            ss = sub * LANES
            cur = [
                [mbuf[wv, k, pl.ds(ss, LANES)] for k in range(K)]
                for wv in range(TPB)
            ]
            while len(cur) > 1:
                cur = [
                    _merge_sorted(cur[2 * i], cur[2 * i + 1])
                    for i in range(len(cur) // 2)
                ]
            fin = cur[0]
            for k in range(K):
                outv[k, pl.ds(ss, LANES)] = fin[k]
            return carry

        lax.fori_loop(0, SUBS, sub_body, 0)
        pltpu.sync_copy(
            outv, out_hbm.at[b, pl.ds(0, K), pl.ds(mcb * 128, 128)]
        )


def kernel(inputs):
    return _topk_sc(inputs)


# speedup vs baseline: 1.8158x; 1.0007x over previous
import functools

import jax
import jax.numpy as jnp
from jax import lax
from jax.experimental import pallas as pl
from jax.experimental.pallas import tpu as pltpu
from jax.experimental.pallas import tpu_sc as plsc

B = 4
L = 8192
C = 768
K = 8
LANES = 16
NCB = 6
SUBS = 8
TPB = 8
LPT = L // TPB
LC = 64
NCHUNK = LPT // LC

_SORT8 = (
    (0, 1), (2, 3), (4, 5), (6, 7),
    (0, 2), (1, 3), (4, 6), (5, 7),
    (1, 2), (5, 6),
    (0, 4), (1, 5), (2, 6), (3, 7),
    (2, 4), (3, 5),
    (1, 2), (3, 4), (5, 6),
)
_BMERGE = (
    (0, 4), (1, 5), (2, 6), (3, 7),
    (0, 2), (1, 3), (4, 6), (5, 7),
    (0, 1), (2, 3), (4, 5), (6, 7),
)


def _merge_sorted(A, Bs):
    M = [jnp.maximum(A[k], Bs[K - 1 - k]) for k in range(K)]
    for a, b in _BMERGE:
        hi = jnp.maximum(M[a], M[b])
        lo = jnp.minimum(M[a], M[b])
        M[a], M[b] = hi, lo
    return M


@functools.partial(
    pl.kernel,
    mesh=plsc.VectorSubcoreMesh(core_axis_name="c", subcore_axis_name="s"),
    out_type=jax.ShapeDtypeStruct((B, K, C), jnp.float32),
    scratch_types=[
        pltpu.VMEM((LC, C), jnp.float32),
        pltpu.VMEM((LC, C), jnp.float32),
        pltpu.VMEM((NCB, K, 128), jnp.float32),
        pltpu.VMEM((TPB, K, 128), jnp.float32),
        pltpu.VMEM((K, 128), jnp.float32),
        pltpu.VMEM_SHARED((NCB, 2 * TPB, K, 128), jnp.float32),
        pltpu.SemaphoreType.DMA,
        pltpu.SemaphoreType.DMA,
    ],
    compiler_params=pltpu.CompilerParams(
        use_tc_tiling_on_sc=True, needs_layout_passes=False
    ),
)
def _topk_sc(x_hbm, out_hbm, buf0, buf1, state, mbuf, outv, S, sem0, sem1):
    core = lax.axis_index("c")
    sid = lax.axis_index("s")
    grp = sid // TPB
    b = 2 * core + grp
    w = sid - grp * TPB
    l0 = w * LPT

    bufs = (buf0, buf1)
    sems = (sem0, sem1)

    def src(chunk):
        return x_hbm.at[b, pl.ds(l0 + chunk * LC, LC)]

    def start(chunk, slot):
        pltpu.async_copy(src(chunk), bufs[slot], sems[slot])

    def wait(chunk, slot):
        pltpu.make_async_copy(src(chunk), bufs[slot], sems[slot]).wait()

    neg_inf = jnp.full((LANES,), -jnp.inf, dtype=jnp.float32)

    @plsc.parallel_loop(0, NCB * K)
    def _init(i):
        cb = i // K
        k = i - cb * K
        for j in range(SUBS):
            state[cb, k, pl.ds(j * LANES, LANES)] = neg_inf

    def process(buf):
        for cb in range(NCB):
            def jbody(j, carry, buf=buf, cb=cb):
                c0 = cb * 128 + j * LANES
                js = j * LANES
                V = tuple(state[cb, k, pl.ds(js, LANES)] for k in range(K))

                @plsc.parallel_loop(0, LC // 8, unroll=2, carry=V)
                def mloop(m, V):
                    rows = [buf[m * 8 + s, pl.ds(c0, LANES)] for s in range(8)]
                    for a, bb in _SORT8:
                        hi = jnp.maximum(rows[a], rows[bb])
                        lo = jnp.minimum(rows[a], rows[bb])
                        rows[a], rows[bb] = hi, lo
                    return tuple(_merge_sorted(V, rows))

                for k in range(K):
                    state[cb, k, pl.ds(js, LANES)] = mloop[k]
                return carry

            lax.fori_loop(0, SUBS, jbody, 0)

    start(0, 0)

    def pair_body(p, carry):
        ch = 2 * p
        start(ch + 1, 1)
        wait(ch, 0)
        process(buf0)

        @pl.when(p < NCHUNK // 2 - 1)
        def _():
            start(ch + 2, 0)

        wait(ch + 1, 1)
        process(buf1)
        return carry

    lax.fori_loop(0, NCHUNK // 2, pair_body, 0)

    for cb in range(NCB):
        pltpu.sync_copy(state.at[cb], S.at[cb, sid])
    plsc.subcore_barrier()

    mcb = sid - grp * TPB

    @pl.when(mcb < NCB)
    def _():
        pltpu.sync_copy(S.at[mcb, pl.ds(grp * TPB, TPB)], mbuf)

        def sub_body(sub, carry):
            ss = sub * LANES
            cur = [
                [mbuf[wv, k, pl.ds(ss, LANES)] for k in range(K)]
                for wv in range(TPB)
            ]
            while len(cur) > 1:
                cur = [
                    _merge_sorted(cur[2 * i], cur[2 * i + 1])
                    for i in range(len(cur) // 2)
                ]
            fin = cur[0]
            for k in range(K):
                outv[k, pl.ds(ss, LANES)] = fin[k]
            return carry

        lax.fori_loop(0, SUBS, sub_body, 0)
        pltpu.sync_copy(
            outv, out_hbm.at[b, pl.ds(0, K), pl.ds(mcb * 128, 128)]
        )


def kernel(inputs):
    return _topk_sc(inputs)
